# Initial kernel scaffold; baseline (speedup 1.0000x reference)
import jax
import jax.numpy as jnp
from jax.experimental import pallas as pl


def kernel(x, edge_index, edge_attr, pos, batch, params):
    raise NotImplementedError("write your pallas kernel here")


# speedup vs baseline: 2.1016x; 2.1016x over previous
import functools

import jax
import jax.numpy as jnp
from jax import lax
from jax.experimental import pallas as pl
from jax.experimental.pallas import tpu as pltpu
from jax.experimental.pallas import tpu_sc as plsc

H = 128
CH = 128
NPAD = 12288
EPAD = 327680
NCHUNK = EPAD // CH
CPT = NCHUNK // 16
CPA = NCHUNK // 32
RPT = NPAD // 16
DEGW = 16

_mesh = plsc.VectorSubcoreMesh(core_axis_name="c", subcore_axis_name="s")
_sc_params = pltpu.CompilerParams(use_tc_tiling_on_sc=False)


def _elu(t):
    return jnp.where(t > 0, t, jnp.exp(jnp.minimum(t, 0.0)) - 1.0)


def _ln_res(base, t, g_ref, b_ref):
    m = jnp.mean(t, axis=-1, keepdims=True)
    v = jnp.mean((t - m) ** 2, axis=-1, keepdims=True)
    return base + (t - m) * lax.rsqrt(v + 1e-5) * g_ref[...] + b_ref[...]


def _gather_body(u_hbm, v_hbm, col2_hbm, row2_hbm, gu_hbm, gv_hbm,
                 idx_slab, rows_v, sem):
    c = lax.axis_index("c")
    s = lax.axis_index("s")

    def run(table, idx2, out):
        base = s * CPT
        pltpu.sync_copy(idx2.at[pl.ds(base, CPT)], idx_slab)

        def step(j, carry):
            pltpu.async_copy(table.at[idx_slab.at[j]], rows_v, sem).wait()
            pltpu.sync_copy(rows_v, out.at[pl.ds((base + j) * CH, CH)])
            return carry

        lax.fori_loop(0, CPT, step, 0)

    @pl.when(c == 0)
    def _():
        run(u_hbm, col2_hbm, gu_hbm)

    @pl.when(c == 1)
    def _():
        run(v_hbm, row2_hbm, gv_hbm)


def _agg_body(eh_hbm, col2_hbm, zeros_hbm, p_hbm,
              shared, big, ehbuf, idxbuf):
    c = lax.axis_index("c")
    s = lax.axis_index("s")
    cb = c * (H // 2)
    pltpu.sync_copy(zeros_hbm, big)
    pltpu.sync_copy(big, shared.at[pl.ds(s * RPT, RPT)])
    plsc.subcore_barrier()
    base = s * CPT

    def step(j, carry):
        ch = base + j
        pltpu.sync_copy(col2_hbm.at[ch], idxbuf)
        pltpu.sync_copy(eh_hbm.at[pl.ds(ch * CH, CH), pl.ds(cb, H // 2)],
                        ehbuf)
        pltpu.sync_copy(ehbuf, shared.at[idxbuf], add=True)
        return carry

    lax.fori_loop(0, CPT, step, 0)
    plsc.subcore_barrier()
    pltpu.sync_copy(shared.at[pl.ds(s * RPT, RPT)], big)
    pltpu.sync_copy(big, p_hbm.at[pl.ds(s * RPT, RPT), pl.ds(cb, H // 2)])


def _deg_body(col2_hbm, zeros8_hbm, ones8_hbm, d0_hbm, d1_hbm,
              shared, big, onesbuf, idxbuf):
    c = lax.axis_index("c")
    s = lax.axis_index("s")
    pltpu.sync_copy(ones8_hbm, onesbuf)
    pltpu.sync_copy(zeros8_hbm, big)
    pltpu.sync_copy(big, shared.at[pl.ds(s * RPT, RPT)])
    plsc.subcore_barrier()
    base = (c * 16 + s) * CPA

    def step(j, carry):
        pltpu.sync_copy(col2_hbm.at[base + j], idxbuf)
        pltpu.sync_copy(onesbuf, shared.at[idxbuf], add=True)
        return carry

    lax.fori_loop(0, CPA, step, 0)
    plsc.subcore_barrier()
    pltpu.sync_copy(shared.at[pl.ds(s * RPT, RPT)], big)

    @pl.when(c == 0)
    def _():
        pltpu.sync_copy(big, d0_hbm.at[pl.ds(s * RPT, RPT)])

    @pl.when(c == 1)
    def _():
        pltpu.sync_copy(big, d1_hbm.at[pl.ds(s * RPT, RPT)])


_gather = pl.kernel(
    _gather_body,
    out_type=(jax.ShapeDtypeStruct((EPAD, H), jnp.float32),
              jax.ShapeDtypeStruct((EPAD, H), jnp.float32)),
    mesh=_mesh,
    compiler_params=_sc_params,
    scratch_types=[
        pltpu.VMEM((CPT, CH), jnp.int32),
        pltpu.VMEM((CH, H), jnp.float32),
        pltpu.SemaphoreType.DMA,
    ],
)

_agg = pl.kernel(
    _agg_body,
    out_type=jax.ShapeDtypeStruct((NPAD, H), jnp.float32),
    mesh=_mesh,
    compiler_params=_sc_params,
    scratch_types=[
        pltpu.VMEM_SHARED((NPAD, H // 2), jnp.float32),
        pltpu.VMEM((RPT, H // 2), jnp.float32),
        pltpu.VMEM((CH, H // 2), jnp.float32),
        pltpu.VMEM((CH,), jnp.int32),
    ],
)

_deg = pl.kernel(
    _deg_body,
    out_type=(jax.ShapeDtypeStruct((NPAD, DEGW), jnp.float32),
              jax.ShapeDtypeStruct((NPAD, DEGW), jnp.float32)),
    mesh=_mesh,
    compiler_params=_sc_params,
    scratch_types=[
        pltpu.VMEM_SHARED((NPAD, DEGW), jnp.float32),
        pltpu.VMEM((RPT, DEGW), jnp.float32),
        pltpu.VMEM((CH, DEGW), jnp.float32),
        pltpu.VMEM((CH,), jnp.int32),
    ],
)


BN = 2048
BE = 2048


def _full(shape=None):
    return pl.BlockSpec(shape, lambda i: (0, 0)) if shape else pl.BlockSpec(
        (1, H), lambda i: (0, 0))


def _rows(bshape):
    return pl.BlockSpec(bshape, lambda i: (i, 0))


def _encode_body(x_ref, w1, b1, w2, b2, g, b, wu, wv,
                 xh_out, u_out, v_out):
    h = _elu(jnp.dot(x_ref[...], w1[...], preferred_element_type=jnp.float32)
             + b1[...])
    t = jnp.dot(h, w2[...], preferred_element_type=jnp.float32) + b2[...]
    xh = _ln_res(jnp.zeros_like(t), t, g, b)
    xh_out[...] = xh
    u_out[...] = jnp.dot(xh, wu[...], preferred_element_type=jnp.float32)
    v_out[...] = jnp.dot(xh, wv[...], preferred_element_type=jnp.float32)


def _edge0_body(ea_ref, gu_ref, gv_ref,
                we1, be1, we2, be2, ge, bbe,
                w1c, b1, w2, b2, g, b, out_ref):
    h = _elu(jnp.dot(ea_ref[...], we1[...], preferred_element_type=jnp.float32)
             + be1[...])
    t = jnp.dot(h, we2[...], preferred_element_type=jnp.float32) + be2[...]
    eh = _ln_res(jnp.zeros_like(t), t, ge, bbe)
    t1 = (gu_ref[...] + gv_ref[...] + b1[...]
          + jnp.dot(eh, w1c[...], preferred_element_type=jnp.float32))
    h1 = _elu(t1)
    t2 = jnp.dot(h1, w2[...], preferred_element_type=jnp.float32) + b2[...]
    out_ref[...] = _ln_res(eh, t2, g, b)


def _edge_body(eh_ref, gu_ref, gv_ref, w1c, b1, w2, b2, g, b, out_ref):
    eh = eh_ref[...]
    t1 = (gu_ref[...] + gv_ref[...] + b1[...]
          + jnp.dot(eh, w1c[...], preferred_element_type=jnp.float32))
    h1 = _elu(t1)
    t2 = jnp.dot(h1, w2[...], preferred_element_type=jnp.float32) + b2[...]
    out_ref[...] = _ln_res(eh, t2, g, b)


def _node_common(xh_ref, pagg, d0, d1, wa, wb, bn1, w2, bn2, g, b):
    deg = jnp.maximum(d0[:, :1] + d1[:, :1], 1.0)
    agg = pagg[...] / deg
    t1 = (jnp.dot(xh_ref[...], wa[...], preferred_element_type=jnp.float32)
          + jnp.dot(agg, wb[...], preferred_element_type=jnp.float32)
          + bn1[...])
    h1 = _elu(t1)
    t2 = jnp.dot(h1, w2[...], preferred_element_type=jnp.float32) + bn2[...]
    return _ln_res(xh_ref[...], t2, g, b)


def _node_body(xh_ref, pagg, d0, d1, wa, wb, bn1, w2, bn2, g, b, wu, wv,
               xh_out, u_out, v_out):
    xh = _node_common(xh_ref, pagg, d0, d1, wa, wb, bn1, w2, bn2, g, b)
    xh_out[...] = xh
    u_out[...] = jnp.dot(xh, wu[...], preferred_element_type=jnp.float32)
    v_out[...] = jnp.dot(xh, wv[...], preferred_element_type=jnp.float32)


def _node_last_body(xh_ref, pagg, d0, d1, wa, wb, bn1, w2, bn2, g, b,
                    wd1, bd1, wd2, bd2, out_ref):
    xh = _node_common(xh_ref, pagg, d0, d1, wa, wb, bn1, w2, bn2, g, b)
    h = _elu(jnp.dot(xh, wd1[...], preferred_element_type=jnp.float32)
             + bd1[...])
    out_ref[...] = jnp.dot(h, wd2[...], preferred_element_type=jnp.float32) \
        + bd2[...]


def _tc(body, grid, in_specs, out_specs, out_shape):
    return pl.pallas_call(body, grid=(grid,), in_specs=in_specs,
                          out_specs=out_specs, out_shape=out_shape)


def kernel(x, edge_index, edge_attr, pos, batch, params):
    N = x.shape[0]
    E = edge_index.shape[1]
    f32 = jnp.float32

    p = params
    wmat = _full((H, H))
    wvec = _full()
    f32s = jax.ShapeDtypeStruct

    def b2d(a):
        return a.reshape(1, H)

    x_p = jnp.zeros((NPAD, H), f32).at[:N].set(x)
    colp = jnp.full((EPAD,), N, jnp.int32).at[:E].set(edge_index[1])
    rowp = jnp.full((EPAD,), N, jnp.int32).at[:E].set(edge_index[0])
    col2 = colp.reshape(NCHUNK, CH)
    row2 = rowp.reshape(NCHUNK, CH)
    ea_p = jnp.zeros((EPAD, edge_attr.shape[1]), f32).at[:E].set(edge_attr)
    zeros_big = jnp.zeros((RPT, H // 2), f32)
    zeros8 = jnp.zeros((RPT, DEGW), f32)
    ones8 = jnp.zeros((CH, DEGW), f32).at[:, 0].set(1.0)

    ew = [dict(w1c=p["edge_mps"][i][0]["W"][2 * H:],
               w1a=p["edge_mps"][i][0]["W"][:H],
               w1b=p["edge_mps"][i][0]["W"][H:2 * H],
               b1=b2d(p["edge_mps"][i][0]["b"]),
               w2=p["edge_mps"][i][1]["W"],
               b2=b2d(p["edge_mps"][i][1]["b"]),
               g=b2d(p["edge_norms"][i]["g"]),
               b=b2d(p["edge_norms"][i]["b"])) for i in range(4)]
    nw = [dict(wa=p["node_mps"][i][0]["W"][:H],
               wb=p["node_mps"][i][0]["W"][H:],
               bn1=b2d(p["node_mps"][i][0]["b"]),
               w2=p["node_mps"][i][1]["W"],
               bn2=b2d(p["node_mps"][i][1]["b"]),
               g=b2d(p["node_norms"][i]["g"]),
               b=b2d(p["node_norms"][i]["b"])) for i in range(4)]

    ngrid = NPAD // BN
    egrid = EPAD // BE

    xh, u, v = _tc(
        _encode_body, ngrid,
        [_rows((BN, H))] + [wmat, wvec, wmat, wvec, wvec, wvec, wmat, wmat],
        [_rows((BN, H))] * 3, [f32s((NPAD, H), f32)] * 3,
    )(x_p, p["node_encode"][0]["W"], b2d(p["node_encode"][0]["b"]),
      p["node_encode"][1]["W"], b2d(p["node_encode"][1]["b"]),
      b2d(p["node_encode_norm"]["g"]), b2d(p["node_encode_norm"]["b"]),
      ew[0]["w1a"], ew[0]["w1b"])

    d0, d1 = _deg(col2, zeros8, ones8)

    eh = None
    out = None
    for i in range(4):
        gu, gv = _gather(u, v, col2, row2)
        e = ew[i]
        if i == 0:
            eh = _tc(
                _edge0_body, egrid,
                [_rows((BE, edge_attr.shape[1]))] + [_rows((BE, H))] * 2
                + [pl.BlockSpec((edge_attr.shape[1], H), lambda i: (0, 0)),
                   wvec, wmat, wvec, wvec, wvec,
                   wmat, wvec, wmat, wvec, wvec, wvec],
                _rows((BE, H)), f32s((EPAD, H), f32),
            )(ea_p, gu, gv,
              p["edge_encode"][0]["W"], b2d(p["edge_encode"][0]["b"]),
              p["edge_encode"][1]["W"], b2d(p["edge_encode"][1]["b"]),
              b2d(p["edge_encode_norm"]["g"]), b2d(p["edge_encode_norm"]["b"]),
              e["w1c"], e["b1"], e["w2"], e["b2"], e["g"], e["b"])
        else:
            eh = _tc(
                _edge_body, egrid,
                [_rows((BE, H))] * 3 + [wmat, wvec, wmat, wvec, wvec, wvec],
                _rows((BE, H)), f32s((EPAD, H), f32),
            )(eh, gu, gv, e["w1c"], e["b1"], e["w2"], e["b2"], e["g"], e["b"])

        pagg = _agg(eh, col2, zeros_big)

        n = nw[i]
        if i < 3:
            xh, u, v = _tc(
                _node_body, ngrid,
                [_rows((BN, H))] * 2 + [_rows((BN, DEGW))] * 2
                + [wmat, wmat, wvec, wmat, wvec, wvec, wvec, wmat, wmat],
                [_rows((BN, H))] * 3, [f32s((NPAD, H), f32)] * 3,
            )(xh, pagg, d0, d1, n["wa"], n["wb"], n["bn1"], n["w2"],
              n["bn2"], n["g"], n["b"], ew[i + 1]["w1a"], ew[i + 1]["w1b"])
        else:
            out = _tc(
                _node_last_body, ngrid,
                [_rows((BN, H))] * 2 + [_rows((BN, DEGW))] * 2
                + [wmat, wmat, wvec, wmat, wvec, wvec, wvec,
                   wmat, wvec, wmat, wvec],
                _rows((BN, H)), f32s((NPAD, H), f32),
            )(xh, pagg, d0, d1, n["wa"], n["wb"], n["bn1"], n["w2"],
              n["bn2"], n["g"], n["b"],
              p["node_decode"][0]["W"], b2d(p["node_decode"][0]["b"]),
              p["node_decode"][1]["W"], b2d(p["node_decode"][1]["b"]))

    return out[:N]


# speedup vs baseline: 2.4218x; 1.1524x over previous
import functools

import jax
import jax.numpy as jnp
from jax import lax
from jax.experimental import pallas as pl
from jax.experimental.pallas import tpu as pltpu
from jax.experimental.pallas import tpu_sc as plsc

H = 128
CH = 128
NPAD = 12288
EPAD = 327680
NCHUNK = EPAD // CH
CPT = NCHUNK // 16
CPA = NCHUNK // 32
RPT = NPAD // 16
DEGW = 16

_mesh = plsc.VectorSubcoreMesh(core_axis_name="c", subcore_axis_name="s")
_sc_params = pltpu.CompilerParams(use_tc_tiling_on_sc=False)


def _elu(t):
    return jnp.where(t > 0, t, jnp.exp(jnp.minimum(t, 0.0)) - 1.0)


def _ln_res(base, t, g_ref, b_ref):
    m = jnp.mean(t, axis=-1, keepdims=True)
    v = jnp.mean((t - m) ** 2, axis=-1, keepdims=True)
    return base + (t - m) * lax.rsqrt(v + 1e-5) * g_ref[...] + b_ref[...]


def _gather_body(u_hbm, v_hbm, col2_hbm, row2_hbm, gu_hbm, gv_hbm,
                 idx_slab, rows0, rows1, gs0, gs1, ws0, ws1):
    c = lax.axis_index("c")
    s = lax.axis_index("s")
    npair = CPT // 2

    def run(table, idx2, out):
        base = s * CPT
        pltpu.sync_copy(idx2.at[pl.ds(base, CPT)], idx_slab)
        pltpu.async_copy(table.at[idx_slab.at[0]], rows0, gs0)

        def pair(k, carry):
            j0 = 2 * k
            j1 = j0 + 1
            pltpu.make_async_copy(table.at[idx_slab.at[j0]], rows0, gs0).wait()

            @pl.when(k > 0)
            def _():
                pltpu.make_async_copy(rows1, out.at[pl.ds(0, CH)], ws1).wait()

            pltpu.async_copy(table.at[idx_slab.at[j1]], rows1, gs1)
            pltpu.async_copy(rows0, out.at[pl.ds((base + j0) * CH, CH)], ws0)
            pltpu.make_async_copy(table.at[idx_slab.at[j1]], rows1, gs1).wait()

            @pl.when(k < npair - 1)
            def _():
                pltpu.make_async_copy(rows0, out.at[pl.ds(0, CH)], ws0).wait()
                pltpu.async_copy(table.at[idx_slab.at[j0 + 2]], rows0, gs0)

            pltpu.async_copy(rows1, out.at[pl.ds((base + j1) * CH, CH)], ws1)
            return carry

        lax.fori_loop(0, npair, pair, 0)
        pltpu.make_async_copy(rows0, out.at[pl.ds(0, CH)], ws0).wait()
        pltpu.make_async_copy(rows1, out.at[pl.ds(0, CH)], ws1).wait()

    @pl.when(c == 0)
    def _():
        run(u_hbm, col2_hbm, gu_hbm)

    @pl.when(c == 1)
    def _():
        run(v_hbm, row2_hbm, gv_hbm)


def _agg_body(eh_hbm, col2_hbm, zeros_hbm, p_hbm,
              shared, big, eh0, eh1, idx0, idx1, ls0, ls1, ss0, ss1):
    c = lax.axis_index("c")
    s = lax.axis_index("s")
    HH = H // 2
    cb = c * HH
    pltpu.sync_copy(zeros_hbm, big)
    pltpu.sync_copy(big, shared.at[pl.ds(s * RPT, RPT)])
    plsc.subcore_barrier()
    base = s * CPT
    npair = CPT // 2

    def ld(j, ib, eb, sem):
        pltpu.async_copy(col2_hbm.at[j], ib, sem)
        pltpu.async_copy(eh_hbm.at[pl.ds(j * CH, CH), pl.ds(cb, HH)], eb, sem)

    def ldwait(j, ib, eb, sem):
        pltpu.make_async_copy(col2_hbm.at[j], ib, sem).wait()
        pltpu.make_async_copy(
            eh_hbm.at[pl.ds(j * CH, CH), pl.ds(cb, HH)], eb, sem).wait()

    ld(base, idx0, eh0, ls0)

    def pair(k, carry):
        j0 = base + 2 * k
        j1 = j0 + 1
        ldwait(j0, idx0, eh0, ls0)

        @pl.when(k > 0)
        def _():
            pltpu.make_async_copy(eh1, shared.at[idx1], ss1).wait()

        ld(j1, idx1, eh1, ls1)
        pltpu.async_copy(eh0, shared.at[idx0], ss0, add=True)
        ldwait(j1, idx1, eh1, ls1)

        @pl.when(k < npair - 1)
        def _():
            pltpu.make_async_copy(eh0, shared.at[idx0], ss0).wait()
            ld(j0 + 2, idx0, eh0, ls0)

        pltpu.async_copy(eh1, shared.at[idx1], ss1, add=True)
        return carry

    lax.fori_loop(0, npair, pair, 0)
    pltpu.make_async_copy(eh0, shared.at[idx0], ss0).wait()
    pltpu.make_async_copy(eh1, shared.at[idx1], ss1).wait()
    plsc.subcore_barrier()
    pltpu.sync_copy(shared.at[pl.ds(s * RPT, RPT)], big)
    pltpu.sync_copy(big, p_hbm.at[pl.ds(s * RPT, RPT), pl.ds(cb, HH)])


def _deg_body(col2_hbm, zeros8_hbm, ones8_hbm, d0_hbm, d1_hbm,
              shared, big, onesbuf, idx0, idx1, ls0, ls1, ss0, ss1):
    c = lax.axis_index("c")
    s = lax.axis_index("s")
    pltpu.sync_copy(ones8_hbm, onesbuf)
    pltpu.sync_copy(zeros8_hbm, big)
    pltpu.sync_copy(big, shared.at[pl.ds(s * RPT, RPT)])
    plsc.subcore_barrier()
    base = (c * 16 + s) * CPA
    npair = CPA // 2
    pltpu.async_copy(col2_hbm.at[base], idx0, ls0)

    def pair(k, carry):
        j0 = base + 2 * k
        j1 = j0 + 1
        pltpu.make_async_copy(col2_hbm.at[j0], idx0, ls0).wait()

        @pl.when(k > 0)
        def _():
            pltpu.make_async_copy(onesbuf, shared.at[idx1], ss1).wait()

        pltpu.async_copy(col2_hbm.at[j1], idx1, ls1)
        pltpu.async_copy(onesbuf, shared.at[idx0], ss0, add=True)
        pltpu.make_async_copy(col2_hbm.at[j1], idx1, ls1).wait()

        @pl.when(k < npair - 1)
        def _():
            pltpu.make_async_copy(onesbuf, shared.at[idx0], ss0).wait()
            pltpu.async_copy(col2_hbm.at[j0 + 2], idx0, ls0)

        pltpu.async_copy(onesbuf, shared.at[idx1], ss1, add=True)
        return carry

    lax.fori_loop(0, npair, pair, 0)
    pltpu.make_async_copy(onesbuf, shared.at[idx0], ss0).wait()
    pltpu.make_async_copy(onesbuf, shared.at[idx1], ss1).wait()
    plsc.subcore_barrier()
    pltpu.sync_copy(shared.at[pl.ds(s * RPT, RPT)], big)

    @pl.when(c == 0)
    def _():
        pltpu.sync_copy(big, d0_hbm.at[pl.ds(s * RPT, RPT)])

    @pl.when(c == 1)
    def _():
        pltpu.sync_copy(big, d1_hbm.at[pl.ds(s * RPT, RPT)])


_gather = pl.kernel(
    _gather_body,
    out_type=(jax.ShapeDtypeStruct((EPAD, H), jnp.float32),
              jax.ShapeDtypeStruct((EPAD, H), jnp.float32)),
    mesh=_mesh,
    compiler_params=_sc_params,
    scratch_types=[
        pltpu.VMEM((CPT, CH), jnp.int32),
        pltpu.VMEM((CH, H), jnp.float32),
        pltpu.VMEM((CH, H), jnp.float32),
        pltpu.SemaphoreType.DMA,
        pltpu.SemaphoreType.DMA,
        pltpu.SemaphoreType.DMA,
        pltpu.SemaphoreType.DMA,
    ],
)

_agg = pl.kernel(
    _agg_body,
    out_type=jax.ShapeDtypeStruct((NPAD, H), jnp.float32),
    mesh=_mesh,
    compiler_params=_sc_params,
    scratch_types=[
        pltpu.VMEM_SHARED((NPAD, H // 2), jnp.float32),
        pltpu.VMEM((RPT, H // 2), jnp.float32),
        pltpu.VMEM((CH, H // 2), jnp.float32),
        pltpu.VMEM((CH, H // 2), jnp.float32),
        pltpu.VMEM((CH,), jnp.int32),
        pltpu.VMEM((CH,), jnp.int32),
        pltpu.SemaphoreType.DMA,
        pltpu.SemaphoreType.DMA,
        pltpu.SemaphoreType.DMA,
        pltpu.SemaphoreType.DMA,
    ],
)

_deg = pl.kernel(
    _deg_body,
    out_type=(jax.ShapeDtypeStruct((NPAD, DEGW), jnp.float32),
              jax.ShapeDtypeStruct((NPAD, DEGW), jnp.float32)),
    mesh=_mesh,
    compiler_params=_sc_params,
    scratch_types=[
        pltpu.VMEM_SHARED((NPAD, DEGW), jnp.float32),
        pltpu.VMEM((RPT, DEGW), jnp.float32),
        pltpu.VMEM((CH, DEGW), jnp.float32),
        pltpu.VMEM((CH,), jnp.int32),
        pltpu.VMEM((CH,), jnp.int32),
        pltpu.SemaphoreType.DMA,
        pltpu.SemaphoreType.DMA,
        pltpu.SemaphoreType.DMA,
        pltpu.SemaphoreType.DMA,
    ],
)


BN = 2048
BE = 2048


def _full(shape=None):
    return pl.BlockSpec(shape, lambda i: (0, 0)) if shape else pl.BlockSpec(
        (1, H), lambda i: (0, 0))


def _rows(bshape):
    return pl.BlockSpec(bshape, lambda i: (i, 0))


def _encode_body(x_ref, w1, b1, w2, b2, g, b, wu, wv,
                 xh_out, u_out, v_out):
    h = _elu(jnp.dot(x_ref[...], w1[...], preferred_element_type=jnp.float32)
             + b1[...])
    t = jnp.dot(h, w2[...], preferred_element_type=jnp.float32) + b2[...]
    xh = _ln_res(jnp.zeros_like(t), t, g, b)
    xh_out[...] = xh
    u_out[...] = jnp.dot(xh, wu[...], preferred_element_type=jnp.float32)
    v_out[...] = jnp.dot(xh, wv[...], preferred_element_type=jnp.float32)


def _edge0_body(ea_ref, gu_ref, gv_ref,
                we1, be1, we2, be2, ge, bbe,
                w1c, b1, w2, b2, g, b, out_ref):
    h = _elu(jnp.dot(ea_ref[...], we1[...], preferred_element_type=jnp.float32)
             + be1[...])
    t = jnp.dot(h, we2[...], preferred_element_type=jnp.float32) + be2[...]
    eh = _ln_res(jnp.zeros_like(t), t, ge, bbe)
    t1 = (gu_ref[...] + gv_ref[...] + b1[...]
          + jnp.dot(eh, w1c[...], preferred_element_type=jnp.float32))
    h1 = _elu(t1)
    t2 = jnp.dot(h1, w2[...], preferred_element_type=jnp.float32) + b2[...]
    out_ref[...] = _ln_res(eh, t2, g, b)


def _edge_body(eh_ref, gu_ref, gv_ref, w1c, b1, w2, b2, g, b, out_ref):
    eh = eh_ref[...]
    t1 = (gu_ref[...] + gv_ref[...] + b1[...]
          + jnp.dot(eh, w1c[...], preferred_element_type=jnp.float32))
    h1 = _elu(t1)
    t2 = jnp.dot(h1, w2[...], preferred_element_type=jnp.float32) + b2[...]
    out_ref[...] = _ln_res(eh, t2, g, b)


def _node_common(xh_ref, pagg, d0, d1, wa, wb, bn1, w2, bn2, g, b):
    deg = jnp.maximum(d0[:, :1] + d1[:, :1], 1.0)
    agg = pagg[...] / deg
    t1 = (jnp.dot(xh_ref[...], wa[...], preferred_element_type=jnp.float32)
          + jnp.dot(agg, wb[...], preferred_element_type=jnp.float32)
          + bn1[...])
    h1 = _elu(t1)
    t2 = jnp.dot(h1, w2[...], preferred_element_type=jnp.float32) + bn2[...]
    return _ln_res(xh_ref[...], t2, g, b)


def _node_body(xh_ref, pagg, d0, d1, wa, wb, bn1, w2, bn2, g, b, wu, wv,
               xh_out, u_out, v_out):
    xh = _node_common(xh_ref, pagg, d0, d1, wa, wb, bn1, w2, bn2, g, b)
    xh_out[...] = xh
    u_out[...] = jnp.dot(xh, wu[...], preferred_element_type=jnp.float32)
    v_out[...] = jnp.dot(xh, wv[...], preferred_element_type=jnp.float32)


def _node_last_body(xh_ref, pagg, d0, d1, wa, wb, bn1, w2, bn2, g, b,
                    wd1, bd1, wd2, bd2, out_ref):
    xh = _node_common(xh_ref, pagg, d0, d1, wa, wb, bn1, w2, bn2, g, b)
    h = _elu(jnp.dot(xh, wd1[...], preferred_element_type=jnp.float32)
             + bd1[...])
    out_ref[...] = jnp.dot(h, wd2[...], preferred_element_type=jnp.float32) \
        + bd2[...]


def _tc(body, grid, in_specs, out_specs, out_shape):
    return pl.pallas_call(body, grid=(grid,), in_specs=in_specs,
                          out_specs=out_specs, out_shape=out_shape)


def kernel(x, edge_index, edge_attr, pos, batch, params):
    N = x.shape[0]
    E = edge_index.shape[1]
    f32 = jnp.float32

    p = params
    wmat = _full((H, H))
    wvec = _full()
    f32s = jax.ShapeDtypeStruct

    def b2d(a):
        return a.reshape(1, H)

    x_p = jnp.zeros((NPAD, H), f32).at[:N].set(x)
    colp = jnp.full((EPAD,), N, jnp.int32).at[:E].set(edge_index[1])
    rowp = jnp.full((EPAD,), N, jnp.int32).at[:E].set(edge_index[0])
    col2 = colp.reshape(NCHUNK, CH)
    row2 = rowp.reshape(NCHUNK, CH)
    ea_p = jnp.zeros((EPAD, edge_attr.shape[1]), f32).at[:E].set(edge_attr)
    zeros_big = jnp.zeros((RPT, H // 2), f32)
    zeros8 = jnp.zeros((RPT, DEGW), f32)
    ones8 = jnp.zeros((CH, DEGW), f32).at[:, 0].set(1.0)

    ew = [dict(w1c=p["edge_mps"][i][0]["W"][2 * H:],
               w1a=p["edge_mps"][i][0]["W"][:H],
               w1b=p["edge_mps"][i][0]["W"][H:2 * H],
               b1=b2d(p["edge_mps"][i][0]["b"]),
               w2=p["edge_mps"][i][1]["W"],
               b2=b2d(p["edge_mps"][i][1]["b"]),
               g=b2d(p["edge_norms"][i]["g"]),
               b=b2d(p["edge_norms"][i]["b"])) for i in range(4)]
    nw = [dict(wa=p["node_mps"][i][0]["W"][:H],
               wb=p["node_mps"][i][0]["W"][H:],
               bn1=b2d(p["node_mps"][i][0]["b"]),
               w2=p["node_mps"][i][1]["W"],
               bn2=b2d(p["node_mps"][i][1]["b"]),
               g=b2d(p["node_norms"][i]["g"]),
               b=b2d(p["node_norms"][i]["b"])) for i in range(4)]

    ngrid = NPAD // BN
    egrid = EPAD // BE

    xh, u, v = _tc(
        _encode_body, ngrid,
        [_rows((BN, H))] + [wmat, wvec, wmat, wvec, wvec, wvec, wmat, wmat],
        [_rows((BN, H))] * 3, [f32s((NPAD, H), f32)] * 3,
    )(x_p, p["node_encode"][0]["W"], b2d(p["node_encode"][0]["b"]),
      p["node_encode"][1]["W"], b2d(p["node_encode"][1]["b"]),
      b2d(p["node_encode_norm"]["g"]), b2d(p["node_encode_norm"]["b"]),
      ew[0]["w1a"], ew[0]["w1b"])

    d0, d1 = _deg(col2, zeros8, ones8)

    eh = None
    out = None
    for i in range(4):
        gu, gv = _gather(u, v, col2, row2)
        e = ew[i]
        if i == 0:
            eh = _tc(
                _edge0_body, egrid,
                [_rows((BE, edge_attr.shape[1]))] + [_rows((BE, H))] * 2
                + [pl.BlockSpec((edge_attr.shape[1], H), lambda i: (0, 0)),
                   wvec, wmat, wvec, wvec, wvec,
                   wmat, wvec, wmat, wvec, wvec, wvec],
                _rows((BE, H)), f32s((EPAD, H), f32),
            )(ea_p, gu, gv,
              p["edge_encode"][0]["W"], b2d(p["edge_encode"][0]["b"]),
              p["edge_encode"][1]["W"], b2d(p["edge_encode"][1]["b"]),
              b2d(p["edge_encode_norm"]["g"]), b2d(p["edge_encode_norm"]["b"]),
              e["w1c"], e["b1"], e["w2"], e["b2"], e["g"], e["b"])
        else:
            eh = _tc(
                _edge_body, egrid,
                [_rows((BE, H))] * 3 + [wmat, wvec, wmat, wvec, wvec, wvec],
                _rows((BE, H)), f32s((EPAD, H), f32),
            )(eh, gu, gv, e["w1c"], e["b1"], e["w2"], e["b2"], e["g"], e["b"])

        pagg = _agg(eh, col2, zeros_big)

        n = nw[i]
        if i < 3:
            xh, u, v = _tc(
                _node_body, ngrid,
                [_rows((BN, H))] * 2 + [_rows((BN, DEGW))] * 2
                + [wmat, wmat, wvec, wmat, wvec, wvec, wvec, wmat, wmat],
                [_rows((BN, H))] * 3, [f32s((NPAD, H), f32)] * 3,
            )(xh, pagg, d0, d1, n["wa"], n["wb"], n["bn1"], n["w2"],
              n["bn2"], n["g"], n["b"], ew[i + 1]["w1a"], ew[i + 1]["w1b"])
        else:
            out = _tc(
                _node_last_body, ngrid,
                [_rows((BN, H))] * 2 + [_rows((BN, DEGW))] * 2
                + [wmat, wmat, wvec, wmat, wvec, wvec, wvec,
                   wmat, wvec, wmat, wvec],
                _rows((BN, H)), f32s((NPAD, H), f32),
            )(xh, pagg, d0, d1, n["wa"], n["wb"], n["bn1"], n["w2"],
              n["bn2"], n["g"], n["b"],
              p["node_decode"][0]["W"], b2d(p["node_decode"][0]["b"]),
              p["node_decode"][1]["W"], b2d(p["node_decode"][1]["b"]))

    return out[:N]
